# baseline (speedup 1.0000x reference)
import functools

import jax
import jax.numpy as jnp
from jax import lax
from jax.experimental import pallas as pl
from jax.experimental.pallas import tpu as pltpu
from jax.experimental.pallas import tpu_sc as plsc

NC = 2
NS = 16
L = 16
NW = NC * NS

V = 1000000
D = 64
CH = D // L
B = 16384
RPW = B // NW
A_PAD = 1024
APW = A_PAD // NW

BLK = 32768
S_SC = 13 * BLK
LPW = S_SC // NW
CW = 256
NCH = LPW // CW
NTCB = (V - S_SC + BLK - 1) // BLK
DROWS = (NTCB + 1) * (BLK // 128)


def _tc_sweep_body(sref, tt_ref, charx_ref, dots_ref, obj_ref):
    i = pl.program_id(0)
    r0 = sref[0]

    @pl.when(i == NTCB)
    def _():
        col = r0 % BLK
        win = tt_ref[:, pl.ds(pl.multiple_of((col // 128) * 128, 128), 128)]
        sel = lax.broadcasted_iota(jnp.int32, (D, 128), 1) == (col % 128)
        obj_ref[...] = jnp.sum(jnp.where(sel, win, 0.0), axis=1, keepdims=True)

    prod = tt_ref[...] * charx_ref[...]
    dots = jnp.sum(prod, axis=0, keepdims=True)
    dots_ref[...] = dots.reshape(BLK // 128, 128)


@jax.jit
def _tc_sweep(tt, charx2, ids_head):
    grid_spec = pltpu.PrefetchScalarGridSpec(
        num_scalar_prefetch=1,
        grid=(NTCB + 1,),
        in_specs=[
            pl.BlockSpec(
                (D, BLK),
                lambda i, s: (0, jnp.where(i < NTCB, i + S_SC // BLK,
                                           s[0] // BLK))),
            pl.BlockSpec((D, 1), lambda i, s: (0, 0)),
        ],
        out_specs=[
            pl.BlockSpec((BLK // 128, 128), lambda i, s: (i, 0)),
            pl.BlockSpec((D, 1), lambda i, s: (0, 0)),
        ],
    )
    return pl.pallas_call(
        _tc_sweep_body,
        grid_spec=grid_spec,
        out_shape=[
            jax.ShapeDtypeStruct((DROWS, 128), jnp.float32),
            jax.ShapeDtypeStruct((D, 1), jnp.float32),
        ],
    )(ids_head, tt, charx2)


def _sc_sweep_body(x3, charb, out_dots,
                   buf0, buf1, charb_v, dst_v, sem0, sem1):
    w = lax.axis_index("s") * NC + lax.axis_index("c")
    base = w * LPW
    pltpu.sync_copy(charb, charb_v)

    bufs = (buf0, buf1)
    sems = (sem0, sem1)

    def fire(chunk, b):
        src = x3.at[:, :, pl.ds(base + chunk * CW, CW)]
        return pltpu.async_copy(src, bufs[b], sems[b])

    fire(0, 0)

    def step(it, carry):
        for bpar in range(2):
            gi = it * 2 + bpar
            pltpu.make_async_copy(
                x3.at[:, :, pl.ds(0, CW)], bufs[bpar], sems[bpar]).wait()
            nxt = jnp.minimum(gi + 1, NCH - 1)
            src = x3.at[:, :, pl.ds(base + nxt * CW, CW)]
            pltpu.async_copy(src, bufs[1 - bpar], sems[1 - bpar])
            buf = bufs[bpar]
            for sub in range(CW // 256):
                accs = [jnp.zeros((L,), jnp.float32) for _ in range(16)]
                for cg in range(8):
                    for cs in range(8):
                        c = cg * 8 + cs
                        cb = charb_v[pl.ds(c * L, L)]
                        for lg in range(16):
                            accs[lg] = accs[lg] + buf[
                                cg, cs, pl.ds(sub * 256 + lg * L, L)] * cb
                for lg in range(16):
                    off = gi * CW + sub * 256 + lg * L
                    dst_v[pl.ds(pl.multiple_of(off, L), L)] = accs[lg]
        return carry

    lax.fori_loop(0, NCH // 2, step, jnp.int32(0))
    pltpu.make_async_copy(x3.at[:, :, pl.ds(0, CW)], bufs[0], sems[0]).wait()
    pltpu.sync_copy(dst_v, out_dots.at[pl.ds(base, LPW)])


@jax.jit
def _sc_sweep(x3, charb):
    mesh = plsc.VectorSubcoreMesh(
        core_axis_name="c", subcore_axis_name="s",
        num_cores=NC, num_subcores=NS,
    )
    call = functools.partial(
        pl.kernel,
        out_type=jax.ShapeDtypeStruct((S_SC,), jnp.float32),
        mesh=mesh,
        compiler_params=pltpu.CompilerParams(
            needs_layout_passes=False, use_tc_tiling_on_sc=True),
        scratch_types=[
            pltpu.VMEM((8, 8, CW), jnp.float32),
            pltpu.VMEM((8, 8, CW), jnp.float32),
            pltpu.VMEM((D * L,), jnp.float32),
            pltpu.VMEM((LPW,), jnp.float32),
            pltpu.SemaphoreType.DMA,
            pltpu.SemaphoreType.DMA,
        ],
    )(_sc_sweep_body)
    return call(x3, charb)


def _dot16(rows_ref, base, cc):
    lane = lax.iota(jnp.int32, L)
    acc = jnp.zeros((L,), jnp.float32)
    for i in range(L):
        r = base + i
        p = rows_ref[r, pl.ds(0, L)] * cc[0]
        for c in range(1, len(cc)):
            p = p + rows_ref[r, pl.ds(c * L, L)] * cc[c]
        acc = jnp.where(lane == i, jnp.sum(p), acc)
    return acc


def _sc_gather_body(dots_sc, dots_tc, atable, w1t, b1, w2t, b2,
                    idx2, act2, charc, objc,
                    out_logit, out_act,
                    idx_v, isc_v, itc_v, vsc_v, vtc_v, char_v, obj_v,
                    w1t_v, b1_v, w2t_v, b2_v,
                    aidx_v, arows_v, lgt_v, aout_v,
                    sem_sc, sem_tc, sem_act):
    w = lax.axis_index("s") * NC + lax.axis_index("c")

    pltpu.sync_copy(idx2.at[pl.ds(w * 4, 4)], idx_v)
    for k in range(4):
        for j in range(8):
            sl = pl.ds(j * L, L)
            ids = idx_v[k, sl]
            isc_v[k, sl] = jnp.minimum(ids, S_SC - 1)
            itc_v[k, sl] = jnp.maximum(ids - S_SC, 0)
    cps = []
    for k in range(4):
        cps.append(pltpu.async_copy(
            dots_sc.at[isc_v.at[k]], vsc_v.at[pl.ds(k * 128, 128)], sem_sc))
        cps.append(pltpu.async_copy(
            dots_tc.at[itc_v.at[k]], vtc_v.at[pl.ds(k * 128, 128)], sem_tc))

    pltpu.sync_copy(act2.at[pl.ds(w, 1)], aidx_v)
    act_cp = pltpu.async_copy(atable.at[aidx_v.at[0]], arows_v, sem_act)

    pltpu.sync_copy(charc, char_v)
    pltpu.sync_copy(objc, obj_v)
    pltpu.sync_copy(w1t, w1t_v)
    pltpu.sync_copy(b1, b1_v)
    pltpu.sync_copy(w2t, w2t_v)
    pltpu.sync_copy(b2, b2_v)

    cat = ([char_v[pl.ds(k * L, L)] for k in range(CH)]
           + [obj_v[pl.ds(k * L, L)] for k in range(CH)])
    h = []
    for jg in range(CH):
        sl = pl.ds(jg * L, L)
        h.append(jnp.maximum(_dot16(w1t_v, jg * L, cat) + b1_v[sl], 0.0))
    ec = []
    for jg in range(CH):
        sl = pl.ds(jg * L, L)
        ec.append(_dot16(w2t_v, jg * L, h) + b2_v[sl])

    act_cp.wait()
    for g in range(APW // L):
        aout_v[pl.ds(g * L, L)] = _dot16(arows_v, g * L, ec)
    pltpu.sync_copy(aout_v, out_act.at[pl.ds(w * APW, APW)])

    for cp in cps:
        cp.wait()
    for g in range(RPW // L):
        sl = pl.ds(g * L, L)
        ids = idx_v[g // 8, pl.ds((g % 8) * L, L)]
        lgt_v[sl] = jnp.where(ids < S_SC, vsc_v[sl], vtc_v[sl])
    pltpu.sync_copy(lgt_v, out_logit.at[pl.ds(w * RPW, RPW)])


@jax.jit
def _sc_gather(dots_sc, dots_tc, atable, w1t, b1, w2t, b2,
               idx2, act2, charc, objc):
    mesh = plsc.VectorSubcoreMesh(
        core_axis_name="c", subcore_axis_name="s",
        num_cores=NC, num_subcores=NS,
    )
    call = functools.partial(
        pl.kernel,
        out_type=(
            jax.ShapeDtypeStruct((B,), jnp.float32),
            jax.ShapeDtypeStruct((A_PAD,), jnp.float32),
        ),
        mesh=mesh,
        compiler_params=pltpu.CompilerParams(
            needs_layout_passes=False, use_tc_tiling_on_sc=False),
        scratch_types=[
            pltpu.VMEM((4, 128), jnp.int32),
            pltpu.VMEM((4, 128), jnp.int32),
            pltpu.VMEM((4, 128), jnp.int32),
            pltpu.VMEM((RPW,), jnp.float32),
            pltpu.VMEM((RPW,), jnp.float32),
            pltpu.VMEM((D,), jnp.float32),
            pltpu.VMEM((D,), jnp.float32),
            pltpu.VMEM((D, 2 * D), jnp.float32),
            pltpu.VMEM((D,), jnp.float32),
            pltpu.VMEM((D, D), jnp.float32),
            pltpu.VMEM((D,), jnp.float32),
            pltpu.VMEM((1, APW), jnp.int32),
            pltpu.VMEM((APW, D), jnp.float32),
            pltpu.VMEM((RPW,), jnp.float32),
            pltpu.VMEM((APW,), jnp.float32),
            pltpu.SemaphoreType.DMA,
            pltpu.SemaphoreType.DMA,
            pltpu.SemaphoreType.DMA,
        ],
    )(_sc_gather_body)
    return call(dots_sc, dots_tc, atable, w1t, b1, w2t, b2,
                idx2, act2, charc, objc)


def kernel(object_table, action_table, W1, b1, W2, b2, node_name_ids, action_ids):
    tt = object_table.T
    x3 = tt.reshape(8, 8, V)
    charx = object_table[0]
    charb = jnp.repeat(charx, L)
    ids_head = node_name_ids[:1]

    dots_sc = _sc_sweep(x3, charb)
    dots_tc, objc = _tc_sweep(tt, charx.reshape(D, 1), ids_head)
    dots_tc = dots_tc.reshape(-1)

    idx2 = node_name_ids.reshape(B // 128, 128)
    act2 = jnp.concatenate(
        [action_ids, jnp.zeros((A_PAD - action_ids.shape[0],), jnp.int32)]
    ).reshape(NW, APW)
    out_logit, out_act = _sc_gather(
        dots_sc, dots_tc, action_table, W1.T, b1, W2.T, b2, idx2, act2,
        charx, objc[:, 0])
    return jnp.concatenate([out_logit, out_act[: action_ids.shape[0]]])


# speedup vs baseline: 2.2607x; 2.2607x over previous
import functools

import jax
import jax.numpy as jnp
from jax import lax
from jax.experimental import pallas as pl
from jax.experimental.pallas import tpu as pltpu
from jax.experimental.pallas import tpu_sc as plsc

NC = 2
NS = 16
L = 16
NW = NC * NS

V = 1000000
D = 64
CH = D // L
B = 16384
RPW = B // NW
A_PAD = 1024
APW = A_PAD // NW

BLK = 32768
NBLK = (V + BLK - 1) // BLK
DROWS = NBLK * (BLK // 128)


def _tc_sweep_body(sref, tt_ref, charx_ref, dots_ref, obj_ref):
    i = pl.program_id(0)
    r0 = sref[0]

    @pl.when(i == r0 // BLK)
    def _():
        col = r0 % BLK
        win = tt_ref[:, pl.ds(pl.multiple_of((col // 128) * 128, 128), 128)]
        sel = lax.broadcasted_iota(jnp.int32, (D, 128), 1) == (col % 128)
        obj_ref[...] = jnp.sum(jnp.where(sel, win, 0.0), axis=1, keepdims=True)

    dots = jax.lax.dot_general(
        charx_ref[...], tt_ref[...], (((1,), (0,)), ((), ())),
        preferred_element_type=jnp.float32)
    dots_ref[...] = dots.reshape(BLK // 128, 128)


@jax.jit
def _tc_sweep(tt, charx_row, ids_head):
    grid_spec = pltpu.PrefetchScalarGridSpec(
        num_scalar_prefetch=1,
        grid=(NBLK,),
        in_specs=[
            pl.BlockSpec((D, BLK), lambda i, s: (0, i)),
            pl.BlockSpec((1, D), lambda i, s: (0, 0)),
        ],
        out_specs=[
            pl.BlockSpec((BLK // 128, 128), lambda i, s: (i, 0)),
            pl.BlockSpec((D, 1), lambda i, s: (0, 0)),
        ],
    )
    return pl.pallas_call(
        _tc_sweep_body,
        grid_spec=grid_spec,
        out_shape=[
            jax.ShapeDtypeStruct((DROWS, 128), jnp.float32),
            jax.ShapeDtypeStruct((D, 1), jnp.float32),
        ],
    )(ids_head, tt, charx_row)


def _dot16(rows_ref, base, cc):
    lane = lax.iota(jnp.int32, L)
    acc = jnp.zeros((L,), jnp.float32)
    for i in range(L):
        r = base + i
        p = rows_ref[r, pl.ds(0, L)] * cc[0]
        for c in range(1, len(cc)):
            p = p + rows_ref[r, pl.ds(c * L, L)] * cc[c]
        acc = jnp.where(lane == i, jnp.sum(p), acc)
    return acc


def _sc_body(dots, atable, w1t, b1, w2t, b2, idx2, act2, charc, objc,
             out_logit, out_act,
             idx_v, char_v, obj_v,
             w1t_v, b1_v, w2t_v, b2_v,
             aidx_v, arows_v, lgt_v, aout_v,
             sem_rows, sem_act):
    w = lax.axis_index("s") * NC + lax.axis_index("c")

    pltpu.sync_copy(idx2.at[pl.ds(w * 4, 4)], idx_v)
    row_cps = [
        pltpu.async_copy(
            dots.at[idx_v.at[k]], lgt_v.at[pl.ds(k * 128, 128)], sem_rows)
        for k in range(4)
    ]

    pltpu.sync_copy(act2.at[pl.ds(w, 1)], aidx_v)
    act_cp = pltpu.async_copy(atable.at[aidx_v.at[0]], arows_v, sem_act)

    pltpu.sync_copy(charc, char_v)
    pltpu.sync_copy(objc, obj_v)
    pltpu.sync_copy(w1t, w1t_v)
    pltpu.sync_copy(b1, b1_v)
    pltpu.sync_copy(w2t, w2t_v)
    pltpu.sync_copy(b2, b2_v)

    cat = ([char_v[pl.ds(k * L, L)] for k in range(CH)]
           + [obj_v[pl.ds(k * L, L)] for k in range(CH)])
    h = []
    for jg in range(CH):
        sl = pl.ds(jg * L, L)
        h.append(jnp.maximum(_dot16(w1t_v, jg * L, cat) + b1_v[sl], 0.0))
    ec = []
    for jg in range(CH):
        sl = pl.ds(jg * L, L)
        ec.append(_dot16(w2t_v, jg * L, h) + b2_v[sl])

    act_cp.wait()
    for g in range(APW // L):
        aout_v[pl.ds(g * L, L)] = _dot16(arows_v, g * L, ec)
    pltpu.sync_copy(aout_v, out_act.at[pl.ds(w * APW, APW)])

    for cp in row_cps:
        cp.wait()
    pltpu.sync_copy(lgt_v, out_logit.at[pl.ds(w * RPW, RPW)])


@jax.jit
def _sc_gather(dots, atable, w1t, b1, w2t, b2, idx2, act2, charc, objc):
    mesh = plsc.VectorSubcoreMesh(
        core_axis_name="c", subcore_axis_name="s",
        num_cores=NC, num_subcores=NS,
    )
    call = functools.partial(
        pl.kernel,
        out_type=(
            jax.ShapeDtypeStruct((B,), jnp.float32),
            jax.ShapeDtypeStruct((A_PAD,), jnp.float32),
        ),
        mesh=mesh,
        compiler_params=pltpu.CompilerParams(
            needs_layout_passes=False, use_tc_tiling_on_sc=False),
        scratch_types=[
            pltpu.VMEM((4, 128), jnp.int32),
            pltpu.VMEM((D,), jnp.float32),
            pltpu.VMEM((D,), jnp.float32),
            pltpu.VMEM((D, 2 * D), jnp.float32),
            pltpu.VMEM((D,), jnp.float32),
            pltpu.VMEM((D, D), jnp.float32),
            pltpu.VMEM((D,), jnp.float32),
            pltpu.VMEM((1, APW), jnp.int32),
            pltpu.VMEM((APW, D), jnp.float32),
            pltpu.VMEM((RPW,), jnp.float32),
            pltpu.VMEM((APW,), jnp.float32),
            pltpu.SemaphoreType.DMA,
            pltpu.SemaphoreType.DMA,
        ],
    )(_sc_body)
    return call(dots, atable, w1t, b1, w2t, b2, idx2, act2, charc, objc)


def kernel(object_table, action_table, W1, b1, W2, b2, node_name_ids, action_ids):
    tt = object_table.T
    charx = object_table[0]
    ids_head = node_name_ids[:1]
    dots, objc = _tc_sweep(tt, charx.reshape(1, D), ids_head)
    dots = dots.reshape(-1)
    idx2 = node_name_ids.reshape(B // 128, 128)
    act2 = jnp.concatenate(
        [action_ids, jnp.zeros((A_PAD - action_ids.shape[0],), jnp.int32)]
    ).reshape(NW, APW)
    out_logit, out_act = _sc_gather(
        dots, action_table, W1.T, b1, W2.T, b2, idx2, act2,
        charx, objc[:, 0])
    return jnp.concatenate([out_logit, out_act[: action_ids.shape[0]]])
